# initial kernel scaffold (unmeasured)
import jax
import jax.numpy as jnp
from jax import lax
from jax.experimental import pallas as pl
from jax.experimental.pallas import tpu as pltpu


def kernel(
    x,
):
    def body(*refs):
        pass

    out_shape = jax.ShapeDtypeStruct(..., jnp.float32)
    return pl.pallas_call(body, out_shape=out_shape)(...)


# baseline (device time: 17970 ns/iter reference)
import jax
import jax.numpy as jnp
from jax import lax
from jax.experimental import pallas as pl
from jax.experimental.pallas import tpu as pltpu

N_DEV = 8


def kernel(x):
    m, n = x.shape

    def body(x_ref, out_ref, tot_ref, recv_ref, send_sems, recv_sems, ack_sem):
        my = lax.axis_index("i")

        tot_ref[0, :] = jnp.sum(x_ref[...], axis=0)

        for j in range(N_DEV):
            for k in range(j + 1, N_DEV):
                @pl.when(my == j)
                def _(j=j, k=k):
                    pltpu.make_async_remote_copy(
                        src_ref=tot_ref,
                        dst_ref=recv_ref.at[j],
                        send_sem=send_sems.at[k],
                        recv_sem=recv_sems.at[j],
                        device_id=(k,),
                        device_id_type=pl.DeviceIdType.MESH,
                    ).start()

        for k in range(N_DEV):
            @pl.when(my == k)
            def _(k=k):
                for j in range(k):
                    pltpu.make_async_remote_copy(
                        src_ref=tot_ref,
                        dst_ref=recv_ref.at[j],
                        send_sem=send_sems.at[k],
                        recv_sem=recv_sems.at[j],
                        device_id=(j,),
                        device_id_type=pl.DeviceIdType.MESH,
                    ).wait_recv()
                    pl.semaphore_signal(
                        ack_sem, inc=1,
                        device_id=(j,),
                        device_id_type=pl.DeviceIdType.MESH,
                    )

        off = jnp.zeros((1, n), dtype=jnp.float32)
        for j in range(N_DEV - 1):
            off = off + jnp.where(j < my, recv_ref[j], 0.0)

        B = 256
        row = lax.broadcasted_iota(jnp.int32, (B, B), 0)
        col = lax.broadcasted_iota(jnp.int32, (B, B), 1)
        tri = (row >= col).astype(jnp.float32)
        carry = off
        for b in range(m // B):
            blk = x_ref[pl.ds(b * B, B), :]
            pref = jnp.dot(tri, blk, preferred_element_type=jnp.float32)
            out_ref[pl.ds(b * B, B), :] = pref + carry
            carry = carry + pref[B - 1 :, :]

        for j in range(N_DEV):
            @pl.when(my == j)
            def _(j=j):
                for k in range(j + 1, N_DEV):
                    pltpu.make_async_remote_copy(
                        src_ref=tot_ref,
                        dst_ref=recv_ref.at[j],
                        send_sem=send_sems.at[k],
                        recv_sem=recv_sems.at[j],
                        device_id=(k,),
                        device_id_type=pl.DeviceIdType.MESH,
                    ).wait_send()
                for _ in range(j + 1, N_DEV):
                    pl.semaphore_wait(ack_sem, 1)

    return pl.pallas_call(
        body,
        out_shape=jax.ShapeDtypeStruct((m, n), x.dtype),
        in_specs=[pl.BlockSpec(memory_space=pltpu.VMEM)],
        out_specs=pl.BlockSpec(memory_space=pltpu.VMEM),
        scratch_shapes=[
            pltpu.VMEM((1, n), x.dtype),
            pltpu.VMEM((N_DEV, 1, n), x.dtype),
            pltpu.SemaphoreType.DMA((N_DEV,)),
            pltpu.SemaphoreType.DMA((N_DEV,)),
            pltpu.SemaphoreType.REGULAR,
        ],
    )(x)


# device time: 17837 ns/iter; 1.0075x vs baseline; 1.0075x over previous
import jax
import jax.numpy as jnp
from jax import lax
from jax.experimental import pallas as pl
from jax.experimental.pallas import tpu as pltpu

N_DEV = 8


def kernel(x):
    m, n = x.shape

    def body(x_ref, out_ref, tot_ref, recv_ref, send_sems, recv_sems, ack_sem):
        my = lax.axis_index("i")

        tot_ref[0, :] = jnp.sum(x_ref[...], axis=0)

        for j in range(N_DEV):
            for k in range(j + 1, N_DEV):
                @pl.when(my == j)
                def _(j=j, k=k):
                    pltpu.make_async_remote_copy(
                        src_ref=tot_ref,
                        dst_ref=recv_ref.at[j],
                        send_sem=send_sems.at[k],
                        recv_sem=recv_sems.at[j],
                        device_id=(k,),
                        device_id_type=pl.DeviceIdType.MESH,
                    ).start()

        B = 128
        row = lax.broadcasted_iota(jnp.int32, (B, B), 0)
        col = lax.broadcasted_iota(jnp.int32, (B, B), 1)
        tri = (row >= col).astype(jnp.float32)
        prefs = []
        for b in range(m // B):
            blk = x_ref[pl.ds(b * B, B), :]
            prefs.append(jnp.dot(tri, blk, preferred_element_type=jnp.float32))

        for k in range(N_DEV):
            @pl.when(my == k)
            def _(k=k):
                for j in range(k):
                    pltpu.make_async_remote_copy(
                        src_ref=tot_ref,
                        dst_ref=recv_ref.at[j],
                        send_sem=send_sems.at[k],
                        recv_sem=recv_sems.at[j],
                        device_id=(j,),
                        device_id_type=pl.DeviceIdType.MESH,
                    ).wait_recv()
                    pl.semaphore_signal(
                        ack_sem, inc=1,
                        device_id=(j,),
                        device_id_type=pl.DeviceIdType.MESH,
                    )

        off = jnp.zeros((1, n), dtype=jnp.float32)
        for j in range(N_DEV - 1):
            off = off + jnp.where(j < my, recv_ref[j], 0.0)

        carry = off
        for b in range(m // B):
            out_ref[pl.ds(b * B, B), :] = prefs[b] + carry
            carry = carry + prefs[b][B - 1 :, :]

        for j in range(N_DEV):
            @pl.when(my == j)
            def _(j=j):
                for k in range(j + 1, N_DEV):
                    pltpu.make_async_remote_copy(
                        src_ref=tot_ref,
                        dst_ref=recv_ref.at[j],
                        send_sem=send_sems.at[k],
                        recv_sem=recv_sems.at[j],
                        device_id=(k,),
                        device_id_type=pl.DeviceIdType.MESH,
                    ).wait_send()
                for _ in range(j + 1, N_DEV):
                    pl.semaphore_wait(ack_sem, 1)

    return pl.pallas_call(
        body,
        out_shape=jax.ShapeDtypeStruct((m, n), x.dtype),
        in_specs=[pl.BlockSpec(memory_space=pltpu.VMEM)],
        out_specs=pl.BlockSpec(memory_space=pltpu.VMEM),
        scratch_shapes=[
            pltpu.VMEM((1, n), x.dtype),
            pltpu.VMEM((N_DEV, 1, n), x.dtype),
            pltpu.SemaphoreType.DMA((N_DEV,)),
            pltpu.SemaphoreType.DMA((N_DEV,)),
            pltpu.SemaphoreType.REGULAR,
        ],
    )(x)
